# baseline (speedup 1.0000x reference)
import functools
import math

import jax
import jax.numpy as jnp
from jax import lax
from jax.experimental import pallas as pl
from jax.experimental.pallas import tpu as pltpu
from jax.experimental.pallas import tpu_sc as plsc

D_MODEL = 128
SCALE = math.sqrt(float(D_MODEL))
NUM_WORKERS = 32
CHUNK = 256
HALF = CHUNK // 2
LANES = 16
NBUF = 2


def _make_kernel(n_rows: int):
    rows_per_worker = n_rows // NUM_WORKERS
    n_chunks = rows_per_worker // CHUNK
    assert rows_per_worker % CHUNK == 0 and n_chunks >= 4
    n_steady = (n_chunks - 4) // NBUF
    n_rem = (n_chunks - 4) % NBUF
    mesh = plsc.VectorSubcoreMesh(core_axis_name="c", subcore_axis_name="s")

    @functools.partial(
        pl.kernel,
        out_type=jax.ShapeDtypeStruct((n_rows, D_MODEL), jnp.float32),
        mesh=mesh,
        scratch_types=[
            pltpu.VMEM((rows_per_worker,), jnp.int32),
            [pltpu.VMEM((CHUNK, D_MODEL), jnp.float32) for _ in range(NBUF)],
            [pltpu.VMEM((HALF, D_MODEL), jnp.float32) for _ in range(2)],
            [pltpu.SemaphoreType.DMA for _ in range(NBUF)],
            [pltpu.SemaphoreType.DMA for _ in range(2)],
        ],
    )
    def gather_scale(x_hbm, table_hbm, out_hbm, idx_v, bin, bout, gsem, ssem):
        wid = lax.axis_index("s") * 2 + lax.axis_index("c")
        base = wid * rows_per_worker
        pltpu.sync_copy(x_hbm.at[pl.ds(base, rows_per_worker)], idx_v)

        def sg(ci, a):
            pltpu.async_copy(table_hbm.at[idx_v.at[pl.ds(ci * CHUNK, CHUNK)]],
                             bin[a], gsem[a])

        def wg(ci, a):
            pltpu.make_async_copy(table_hbm.at[idx_v.at[pl.ds(ci * CHUNK, CHUNK)]],
                                  bin[a], gsem[a]).wait()

        def ss(ci, h):
            pltpu.async_copy(
                bout[h],
                out_hbm.at[pl.ds(base + ci * CHUNK + h * HALF, HALF)], ssem[h])

        def ws(h):
            pltpu.make_async_copy(bout[h], out_hbm.at[pl.ds(base, HALF)],
                                  ssem[h]).wait()

        def scale_half(a, h):
            def row(i, _):
                for j in range(D_MODEL // LANES):
                    sl = pl.ds(j * LANES, LANES)
                    bout[h][i, sl] = bin[a][h * HALF + i, sl] * SCALE
                return 0

            lax.fori_loop(0, HALF, row, 0)

        def body(ci, a, first=False, issue=True):
            wg(ci, a)
            for h in range(2):
                if not first:
                    ws(h)
                scale_half(a, h)
                ss(ci, h)
            if issue:
                sg(ci + NBUF, a)

        sg(0, 0)
        sg(1, 1)
        body(0, 0, first=True)
        body(1, 1)

        def group(g, _):
            ci0 = g * NBUF
            for k in range(NBUF):
                body(ci0 + k, k)
            return 0

        lax.fori_loop(1, 1 + n_steady, group, 0)

        ci0 = (1 + n_steady) * NBUF
        for k in range(n_rem):
            body(ci0 + k, (ci0 + k) % NBUF)
        for k in range(n_rem, n_rem + NBUF):
            ci = ci0 + k
            body(ci, ci % NBUF, issue=False)
        for h in range(2):
            ws(h)

    return gather_scale


def kernel(x, table):
    b, s = x.shape
    n_rows = b * s
    out = _make_kernel(n_rows)(x.reshape(n_rows).astype(jnp.int32), table)
    return out.reshape(b, s, D_MODEL)
